# baseline (speedup 1.0000x reference)
import jax
import jax.numpy as jnp
from jax import lax
from jax.experimental import pallas as pl
from jax.experimental.pallas import tpu as pltpu
from jax.experimental.pallas import tpu_sc as plsc

_R = 128
_N = _R * _R * _R
_B = 2
_TOT = _B * _N
_NW = 32
_VPW = _TOT // _NW
_CH = 1024
_NCHUNK = _VPW // _CH
_STEPS = _CH // 16


def _round_bf16(v):
    c = v * jnp.float32(65537.0)
    return c - (c - v)


def _sc_body(flat_hbm, coef_hbm, out_hbm,
             idx_v, gat_v, wt_v, out_v, coef_v, cidx_v, cv_v, sem):
    wid = lax.axis_index("s") * 2 + lax.axis_index("c")
    b = wid // 16
    base_n = wid * _VPW

    pltpu.sync_copy(coef_hbm.at[pl.ds(b * 16, 16)], coef_v)

    lane = lax.iota(jnp.int32, 16)
    c16 = coef_v[...]
    t00 = c16[0]
    t01 = c16[1]
    t02 = c16[2]
    t03 = c16[3]
    t10 = c16[4]
    t11 = c16[5]
    t12 = c16[6]
    t13 = c16[7]
    t20 = c16[8]
    t21 = c16[9]
    t22 = c16[10]
    t23 = c16[11]
    half = jnp.float32(0.5)
    one = jnp.float32(1.0)
    rf = jnp.float32(_R)
    base_i = b * _N

    xbit = jnp.bitwise_and(lane, 1)
    ybit = jnp.bitwise_and(lax.shift_right_logical(lane, 1), 1)
    zbit = jnp.bitwise_and(lax.shift_right_logical(lane, 2), 1)
    cidx_v[...] = (base_i + xbit * 127 + ybit * 16256 + zbit * 2080768)
    pltpu.async_copy(flat_hbm.at[cidx_v], cv_v, sem).wait()
    cvv = cv_v[...]
    cv0 = cvv[0]
    cv1 = cvv[1]
    cv2 = cvv[2]
    cv3 = cvv[3]
    cv4 = cvv[4]
    cv5 = cvv[5]
    cv6 = cvv[6]
    cv7 = cvv[7]

    def axis_parts(coord):
        ci = coord.astype(jnp.int32)
        c0 = jnp.minimum(jnp.maximum(ci, 0), 127)
        c1 = jnp.minimum(jnp.maximum(ci + 1, 0), 127)
        c0f = c0.astype(jnp.float32)
        c1f = c1.astype(jnp.float32)
        return c0, c1 - c0, c1f - coord, coord - c0f

    def chunk(c, _):
        n0 = base_n + c * _CH

        def compute(s, _):
            n = n0 + s * 16 + lane
            kk = jnp.bitwise_and(n, _R - 1)
            jj = jnp.bitwise_and(lax.shift_right_logical(n, 7), _R - 1)
            ii = jnp.bitwise_and(lax.shift_right_logical(n, 14), _R - 1)
            lstep = jnp.float32(2.0 / 127.0)
            xlj = _round_bf16(jj.astype(jnp.float32) * lstep - one)
            yli = _round_bf16(ii.astype(jnp.float32) * lstep - one)
            zlk = _round_bf16(kk.astype(jnp.float32) * lstep - one)
            xs = (t00 * xlj + t03) + (t01 * yli + t02 * zlk)
            ys = (t10 * xlj + t13) + (t11 * yli + t12 * zlk)
            zs = (t20 * xlj + t23) + (t21 * yli + t22 * zlk)
            xp = half * (xs + one) * rf
            yp = half * (ys + one) * rf
            zp = half * (zs + one) * rf
            x0, dx, ax0, ax1 = axis_parts(xp)
            y0, dy, ay0, ay1 = axis_parts(yp)
            z0, dz, az0, az1 = axis_parts(zp)
            i000 = (base_i + lax.shift_left(z0, 14)
                    + lax.shift_left(y0, 7) + x0)
            dyw = lax.shift_left(dy, 7)
            dzw = lax.shift_left(dz, 14)
            deg = (dx + (dy + dz)) == 0
            xhiv = x0 == 127
            yhiv = y0 == 127
            zhiv = z0 == 127
            v01 = jnp.where(xhiv, cv1, cv0)
            v23 = jnp.where(xhiv, cv3, cv2)
            v45 = jnp.where(xhiv, cv5, cv4)
            v67 = jnp.where(xhiv, cv7, cv6)
            va = jnp.where(yhiv, v23, v01)
            vb = jnp.where(yhiv, v67, v45)
            vsel = jnp.where(zhiv, vb, va)
            dum = jnp.bitwise_and(n, _N - 1)
            i000 = jnp.where(deg, dum, i000)
            cx = dx == 0
            cy = dy == 0
            cz = dz == 0
            cxy = jnp.logical_or(cx, cy)
            cxz = jnp.logical_or(cx, cz)
            cyz = jnp.logical_or(cy, cz)
            cxyz = jnp.logical_or(cxy, cz)
            s16 = s * 16
            idx_v[pl.ds(0 * _CH + s16, 16)] = i000
            idx_v[pl.ds(1 * _CH + s16, 16)] = jnp.where(cz, dum, i000 + dzw)
            idx_v[pl.ds(2 * _CH + s16, 16)] = jnp.where(cy, dum, i000 + dyw)
            idx_v[pl.ds(3 * _CH + s16, 16)] = jnp.where(cyz, dum, i000 + (dyw + dzw))
            idx_v[pl.ds(4 * _CH + s16, 16)] = jnp.where(cx, dum, i000 + dx)
            idx_v[pl.ds(5 * _CH + s16, 16)] = jnp.where(cxz, dum, i000 + (dx + dzw))
            idx_v[pl.ds(6 * _CH + s16, 16)] = jnp.where(cxy, dum, i000 + (dx + dyw))
            idx_v[pl.ds(7 * _CH + s16, 16)] = jnp.where(cxyz, dum, i000 + ((dx + dyw) + dzw))
            wt_v[0, pl.ds(s16, 16)] = ax0
            wt_v[1, pl.ds(s16, 16)] = ax1
            wt_v[2, pl.ds(s16, 16)] = ay0
            wt_v[3, pl.ds(s16, 16)] = ay1
            wt_v[4, pl.ds(s16, 16)] = az0
            wt_v[5, pl.ds(s16, 16)] = az1
            wt_v[6, pl.ds(s16, 16)] = jnp.where(deg, one, jnp.float32(0.0))
            wt_v[7, pl.ds(s16, 16)] = vsel
            zero = jnp.float32(0.0)
            wt_v[8, pl.ds(s16, 16)] = jnp.where(cx, one, zero)
            wt_v[9, pl.ds(s16, 16)] = jnp.where(cy, one, zero)
            wt_v[10, pl.ds(s16, 16)] = jnp.where(cz, one, zero)
            return _

        lax.fori_loop(0, _STEPS, compute, None)

        pltpu.async_copy(flat_hbm.at[idx_v], gat_v, sem).wait()

        def combine(s, _):
            s16 = s * 16
            ax0 = wt_v[0, pl.ds(s16, 16)]
            ax1 = wt_v[1, pl.ds(s16, 16)]
            ay0 = wt_v[2, pl.ds(s16, 16)]
            ay1 = wt_v[3, pl.ds(s16, 16)]
            az0 = wt_v[4, pl.ds(s16, 16)]
            az1 = wt_v[5, pl.ds(s16, 16)]
            degm = wt_v[6, pl.ds(s16, 16)] > half
            vsel = wt_v[7, pl.ds(s16, 16)]
            cx = wt_v[8, pl.ds(s16, 16)] > half
            cy = wt_v[9, pl.ds(s16, 16)] > half
            cz = wt_v[10, pl.ds(s16, 16)] > half
            r000 = gat_v[pl.ds(0 * _CH + s16, 16)]
            r001 = gat_v[pl.ds(1 * _CH + s16, 16)]
            r010 = gat_v[pl.ds(2 * _CH + s16, 16)]
            r011 = gat_v[pl.ds(3 * _CH + s16, 16)]
            r100 = gat_v[pl.ds(4 * _CH + s16, 16)]
            r101 = gat_v[pl.ds(5 * _CH + s16, 16)]
            r110 = gat_v[pl.ds(6 * _CH + s16, 16)]
            r111 = gat_v[pl.ds(7 * _CH + s16, 16)]
            G000 = r000
            G100 = jnp.where(cx, G000, r100)
            G010 = jnp.where(cy, G000, r010)
            G110 = jnp.where(cx, G010, jnp.where(cy, G100, r110))
            G001 = jnp.where(cz, G000, r001)
            G101 = jnp.where(cz, G100, jnp.where(cx, G001, r101))
            G011 = jnp.where(cz, G010, jnp.where(cy, G001, r011))
            G111 = jnp.where(cz, G110, jnp.where(cx, G011, jnp.where(cy, G101, r111)))
            g000 = jnp.where(degm, vsel, G000)
            g001 = jnp.where(degm, vsel, G001)
            g010 = jnp.where(degm, vsel, G010)
            g011 = jnp.where(degm, vsel, G011)
            g100 = jnp.where(degm, vsel, G100)
            g101 = jnp.where(degm, vsel, G101)
            g110 = jnp.where(degm, vsel, G110)
            g111 = jnp.where(degm, vsel, G111)
            m00 = ax0 * ay0
            m01 = ax0 * ay1
            m10 = ax1 * ay0
            m11 = ax1 * ay1
            acc = (m00 * az0) * g000
            acc = acc + (m00 * az1) * g001
            acc = acc + (m01 * az0) * g010
            acc = acc + (m01 * az1) * g011
            acc = acc + (m10 * az0) * g100
            acc = acc + (m10 * az1) * g101
            acc = acc + (m11 * az0) * g110
            acc = acc + (m11 * az1) * g111
            out_v[pl.ds(s16, 16)] = acc
            return _

        lax.fori_loop(0, _STEPS, combine, None)
        pltpu.sync_copy(out_v, out_hbm.at[pl.ds(n0, _CH)])
        return _

    lax.fori_loop(0, _NCHUNK, chunk, None)


@jax.jit
def _run(flat, coef):
    mesh = plsc.VectorSubcoreMesh(core_axis_name="c", subcore_axis_name="s")
    f = pl.kernel(
        _sc_body,
        out_type=jax.ShapeDtypeStruct((_TOT,), jnp.float32),
        mesh=mesh,
        scratch_types=[
            pltpu.VMEM((8 * _CH,), jnp.int32),
            pltpu.VMEM((8 * _CH,), jnp.float32),
            pltpu.VMEM((11, _CH), jnp.float32),
            pltpu.VMEM((_CH,), jnp.float32),
            pltpu.VMEM((16,), jnp.float32),
            pltpu.VMEM((16,), jnp.int32),
            pltpu.VMEM((16,), jnp.float32),
            pltpu.SemaphoreType.DMA,
        ],
    )
    return f(flat, coef)


def kernel(images, transform_parameters):
    B, H, W, D, C = images.shape
    flat = images.reshape(-1)
    T = transform_parameters.reshape(B, 12)
    Tb = T.astype(jnp.bfloat16).astype(jnp.float32)
    coef = jnp.concatenate([Tb, jnp.zeros((B, 4), jnp.float32)], axis=1).reshape(-1)
    out = _run(flat, coef)
    return out.reshape(B, _R, _R, _R, 1)


# speedup vs baseline: 2.3342x; 2.3342x over previous
import jax
import jax.numpy as jnp
from jax import lax
from jax.experimental import pallas as pl
from jax.experimental.pallas import tpu as pltpu
from jax.experimental.pallas import tpu_sc as plsc

_R = 128
_N = _R * _R * _R
_B = 2
_TOT = _B * _N
_NW = 32
_VPW = _TOT // _NW
_CH = 1024
_REP = 64
_FB = _TOT
_NCHUNK = _VPW // _CH
_STEPS = _CH // 16


def _round_bf16(v):
    c = v * jnp.float32(65537.0)
    return c - (c - v)


def _sc_body(flat_hbm, coef_hbm, out_hbm,
             idx_v, gat_v, wt_v, out_v, coef_v, cidx_v, cv_v, sem):
    wid = lax.axis_index("s") * 2 + lax.axis_index("c")
    b = wid // 16
    base_n = wid * _VPW

    pltpu.sync_copy(coef_hbm.at[pl.ds(b * 16, 16)], coef_v)

    lane = lax.iota(jnp.int32, 16)
    c16 = coef_v[...]
    t00 = c16[0]
    t01 = c16[1]
    t02 = c16[2]
    t03 = c16[3]
    t10 = c16[4]
    t11 = c16[5]
    t12 = c16[6]
    t13 = c16[7]
    t20 = c16[8]
    t21 = c16[9]
    t22 = c16[10]
    t23 = c16[11]
    half = jnp.float32(0.5)
    one = jnp.float32(1.0)
    rf = jnp.float32(_R)
    base_i = b * _N

    xbit = jnp.bitwise_and(lane, 1)
    ybit = jnp.bitwise_and(lax.shift_right_logical(lane, 1), 1)
    zbit = jnp.bitwise_and(lax.shift_right_logical(lane, 2), 1)
    cidx_v[...] = (base_i + xbit * 127 + ybit * 16256 + zbit * 2080768)
    pltpu.async_copy(flat_hbm.at[cidx_v], cv_v, sem).wait()
    cvv = cv_v[...]
    cv0 = cvv[0]
    cv1 = cvv[1]
    cv2 = cvv[2]
    cv3 = cvv[3]
    cv4 = cvv[4]
    cv5 = cvv[5]
    cv6 = cvv[6]
    cv7 = cvv[7]

    def axis_parts(coord):
        ci = coord.astype(jnp.int32)
        c0 = jnp.minimum(jnp.maximum(ci, 0), 127)
        c1 = jnp.minimum(jnp.maximum(ci + 1, 0), 127)
        c0f = c0.astype(jnp.float32)
        c1f = c1.astype(jnp.float32)
        return c0, c1 - c0, c1f - coord, coord - c0f

    def chunk(c, _):
        n0 = base_n + c * _CH

        def compute(s, _):
            n = n0 + s * 16 + lane
            kk = jnp.bitwise_and(n, _R - 1)
            jj = jnp.bitwise_and(lax.shift_right_logical(n, 7), _R - 1)
            ii = jnp.bitwise_and(lax.shift_right_logical(n, 14), _R - 1)
            lstep = jnp.float32(2.0 / 127.0)
            xlj = _round_bf16(jj.astype(jnp.float32) * lstep - one)
            yli = _round_bf16(ii.astype(jnp.float32) * lstep - one)
            zlk = _round_bf16(kk.astype(jnp.float32) * lstep - one)
            xs = (t00 * xlj + t03) + (t01 * yli + t02 * zlk)
            ys = (t10 * xlj + t13) + (t11 * yli + t12 * zlk)
            zs = (t20 * xlj + t23) + (t21 * yli + t22 * zlk)
            xp = half * (xs + one) * rf
            yp = half * (ys + one) * rf
            zp = half * (zs + one) * rf
            x0, dx, ax0, ax1 = axis_parts(xp)
            y0, dy, ay0, ay1 = axis_parts(yp)
            z0, dz, az0, az1 = axis_parts(zp)
            i000 = (base_i + lax.shift_left(z0, 14)
                    + lax.shift_left(y0, 7) + x0)
            dyw = lax.shift_left(dy, 7)
            dzw = lax.shift_left(dz, 14)
            deg = (dx + (dy + dz)) == 0
            xhiv = x0 == 127
            yhiv = y0 == 127
            zhiv = z0 == 127
            v01 = jnp.where(xhiv, cv1, cv0)
            v23 = jnp.where(xhiv, cv3, cv2)
            v45 = jnp.where(xhiv, cv5, cv4)
            v67 = jnp.where(xhiv, cv7, cv6)
            va = jnp.where(yhiv, v23, v01)
            vb = jnp.where(yhiv, v67, v45)
            vsel = jnp.where(zhiv, vb, va)
            dum = jnp.bitwise_and(n, _N - 1)
            cx = dx == 0
            cy = dy == 0
            cz = dz == 0
            cxy = jnp.logical_or(cx, cy)
            cxz = jnp.logical_or(cx, cz)
            cyz = jnp.logical_or(cy, cz)
            cxyz = jnp.logical_or(cxy, cz)
            anyc = cxyz
            xhi_i = jnp.where(xhiv, 1, 0)
            yhi_i = jnp.where(yhiv, 1, 0)
            zhi_i = jnp.where(zhiv, 1, 0)
            fid = jnp.where(cx, xhi_i, jnp.where(cy, 2 + yhi_i, 4 + zhi_i))
            rep = jnp.bitwise_and(n, _REP - 1)
            fbase = (_FB + lax.shift_left(rep * 12 + (b * 6 + fid), 14))
            z07 = lax.shift_left(z0, 7)
            dz7 = lax.shift_left(dz, 7)
            y07 = lax.shift_left(y0, 7)
            dy7 = lax.shift_left(dy, 7)
            def fpos(zz7, yy, yy7, xx):
                return jnp.where(cx, zz7 + yy, jnp.where(cy, zz7 + xx, yy7 + xx))
            x1c = x0 + dx
            y1c = y0 + dy
            p000 = fpos(z07, y0, y07, x0)
            p001 = fpos(z07 + dz7, y0, y07, x0)
            p010 = fpos(z07, y1c, y07 + dy7, x0)
            p011 = fpos(z07 + dz7, y1c, y07 + dy7, x0)
            p100 = fpos(z07, y0, y07, x1c)
            p101 = fpos(z07 + dz7, y0, y07, x1c)
            p110 = fpos(z07, y1c, y07 + dy7, x1c)
            p111 = fpos(z07 + dz7, y1c, y07 + dy7, x1c)
            def route(used_dum, flat_i, pos):
                real = jnp.where(anyc, fbase + pos, flat_i)
                return jnp.where(used_dum, dum, real)
            s16 = s * 16
            idx_v[pl.ds(0 * _CH + s16, 16)] = route(deg, i000, p000)
            idx_v[pl.ds(1 * _CH + s16, 16)] = route(cz, i000 + dzw, p001)
            idx_v[pl.ds(2 * _CH + s16, 16)] = route(cy, i000 + dyw, p010)
            idx_v[pl.ds(3 * _CH + s16, 16)] = route(cyz, i000 + (dyw + dzw), p011)
            idx_v[pl.ds(4 * _CH + s16, 16)] = route(cx, i000 + dx, p100)
            idx_v[pl.ds(5 * _CH + s16, 16)] = route(cxz, i000 + (dx + dzw), p101)
            idx_v[pl.ds(6 * _CH + s16, 16)] = route(cxy, i000 + (dx + dyw), p110)
            idx_v[pl.ds(7 * _CH + s16, 16)] = route(cxyz, i000 + ((dx + dyw) + dzw), p111)
            wt_v[0, pl.ds(s16, 16)] = ax0
            wt_v[1, pl.ds(s16, 16)] = ax1
            wt_v[2, pl.ds(s16, 16)] = ay0
            wt_v[3, pl.ds(s16, 16)] = ay1
            wt_v[4, pl.ds(s16, 16)] = az0
            wt_v[5, pl.ds(s16, 16)] = az1
            wt_v[6, pl.ds(s16, 16)] = jnp.where(deg, one, jnp.float32(0.0))
            wt_v[7, pl.ds(s16, 16)] = vsel
            zero = jnp.float32(0.0)
            wt_v[8, pl.ds(s16, 16)] = jnp.where(cx, one, zero)
            wt_v[9, pl.ds(s16, 16)] = jnp.where(cy, one, zero)
            wt_v[10, pl.ds(s16, 16)] = jnp.where(cz, one, zero)
            return _

        lax.fori_loop(0, _STEPS, compute, None)

        pltpu.async_copy(flat_hbm.at[idx_v], gat_v, sem).wait()

        def combine(s, _):
            s16 = s * 16
            ax0 = wt_v[0, pl.ds(s16, 16)]
            ax1 = wt_v[1, pl.ds(s16, 16)]
            ay0 = wt_v[2, pl.ds(s16, 16)]
            ay1 = wt_v[3, pl.ds(s16, 16)]
            az0 = wt_v[4, pl.ds(s16, 16)]
            az1 = wt_v[5, pl.ds(s16, 16)]
            degm = wt_v[6, pl.ds(s16, 16)] > half
            vsel = wt_v[7, pl.ds(s16, 16)]
            cx = wt_v[8, pl.ds(s16, 16)] > half
            cy = wt_v[9, pl.ds(s16, 16)] > half
            cz = wt_v[10, pl.ds(s16, 16)] > half
            r000 = gat_v[pl.ds(0 * _CH + s16, 16)]
            r001 = gat_v[pl.ds(1 * _CH + s16, 16)]
            r010 = gat_v[pl.ds(2 * _CH + s16, 16)]
            r011 = gat_v[pl.ds(3 * _CH + s16, 16)]
            r100 = gat_v[pl.ds(4 * _CH + s16, 16)]
            r101 = gat_v[pl.ds(5 * _CH + s16, 16)]
            r110 = gat_v[pl.ds(6 * _CH + s16, 16)]
            r111 = gat_v[pl.ds(7 * _CH + s16, 16)]
            G000 = r000
            G100 = jnp.where(cx, G000, r100)
            G010 = jnp.where(cy, G000, r010)
            G110 = jnp.where(cx, G010, jnp.where(cy, G100, r110))
            G001 = jnp.where(cz, G000, r001)
            G101 = jnp.where(cz, G100, jnp.where(cx, G001, r101))
            G011 = jnp.where(cz, G010, jnp.where(cy, G001, r011))
            G111 = jnp.where(cz, G110, jnp.where(cx, G011, jnp.where(cy, G101, r111)))
            g000 = jnp.where(degm, vsel, G000)
            g001 = jnp.where(degm, vsel, G001)
            g010 = jnp.where(degm, vsel, G010)
            g011 = jnp.where(degm, vsel, G011)
            g100 = jnp.where(degm, vsel, G100)
            g101 = jnp.where(degm, vsel, G101)
            g110 = jnp.where(degm, vsel, G110)
            g111 = jnp.where(degm, vsel, G111)
            m00 = ax0 * ay0
            m01 = ax0 * ay1
            m10 = ax1 * ay0
            m11 = ax1 * ay1
            acc = (m00 * az0) * g000
            acc = acc + (m00 * az1) * g001
            acc = acc + (m01 * az0) * g010
            acc = acc + (m01 * az1) * g011
            acc = acc + (m10 * az0) * g100
            acc = acc + (m10 * az1) * g101
            acc = acc + (m11 * az0) * g110
            acc = acc + (m11 * az1) * g111
            out_v[pl.ds(s16, 16)] = acc
            return _

        lax.fori_loop(0, _STEPS, combine, None)
        pltpu.sync_copy(out_v, out_hbm.at[pl.ds(n0, _CH)])
        return _

    lax.fori_loop(0, _NCHUNK, chunk, None)


@jax.jit
def _run(flat, coef):
    mesh = plsc.VectorSubcoreMesh(core_axis_name="c", subcore_axis_name="s")
    f = pl.kernel(
        _sc_body,
        out_type=jax.ShapeDtypeStruct((_TOT,), jnp.float32),
        mesh=mesh,
        scratch_types=[
            pltpu.VMEM((8 * _CH,), jnp.int32),
            pltpu.VMEM((8 * _CH,), jnp.float32),
            pltpu.VMEM((11, _CH), jnp.float32),
            pltpu.VMEM((_CH,), jnp.float32),
            pltpu.VMEM((16,), jnp.float32),
            pltpu.VMEM((16,), jnp.int32),
            pltpu.VMEM((16,), jnp.float32),
            pltpu.SemaphoreType.DMA,
        ],
    )
    return f(flat, coef)


def kernel(images, transform_parameters):
    B, H, W, D, C = images.shape
    flat = images.reshape(-1)
    vol = images.reshape(B, H, W, D)
    faces = jnp.stack([
        vol[:, :, :, 0], vol[:, :, :, 127],
        vol[:, :, 0, :], vol[:, :, 127, :],
        vol[:, 0, :, :], vol[:, 127, :, :],
    ], axis=1)
    faces_rep = jnp.broadcast_to(faces.reshape(1, B * 6 * 16384),
                                 (_REP, B * 6 * 16384)).reshape(-1)
    big = jnp.concatenate([flat, faces_rep])
    T = transform_parameters.reshape(B, 12)
    Tb = T.astype(jnp.bfloat16).astype(jnp.float32)
    coef = jnp.concatenate([Tb, jnp.zeros((B, 4), jnp.float32)], axis=1).reshape(-1)
    out = _run(big, coef)
    return out.reshape(B, _R, _R, _R, 1)
